# Initial kernel scaffold; baseline (speedup 1.0000x reference)
import jax
import jax.numpy as jnp
from jax import lax
from jax.experimental import pallas as pl
from jax.experimental.pallas import tpu as pltpu
from jax.experimental.pallas import tpu_sc as plsc

VOCAB = 1000000
MAX_TOKENS = 1000
EMBED_DIM = 32
BATCH = 16384
SEQ = 20

NUM_CORES = 2
NUM_SUBCORES = 16
LANES = 16
NUM_WORKERS = NUM_CORES * NUM_SUBCORES
B_PER_W = BATCH // NUM_WORKERS
IDX_CHUNK = 128
N_IDX_CHUNKS = B_PER_W // IDX_CHUNK
ROW_CHUNKS = B_PER_W // LANES


def _body(tid_hbm, tok_hbm, title_tab_hbm, token_tab_hbm, out_hbm,
          tid_v, title_rows_v, ttab_v, tok_v, pool_v, gsem):
    wid = lax.axis_index("s") * NUM_CORES + lax.axis_index("c")
    base = wid * B_PER_W

    pltpu.sync_copy(tid_hbm.at[wid], tid_v)
    gathers = []
    for c in range(N_IDX_CHUNKS):
        gathers.append(
            pltpu.async_copy(
                title_tab_hbm.at[tid_v.at[c]],
                title_rows_v.at[pl.ds(c * IDX_CHUNK, IDX_CHUNK)],
                gsem,
            )
        )

    pltpu.sync_copy(token_tab_hbm, ttab_v)
    pltpu.sync_copy(tok_hbm.at[wid], tok_v)

    zeros16 = jnp.zeros((LANES,), jnp.float32)
    ttab_v[0, pl.ds(0, LANES)] = zeros16
    ttab_v[0, pl.ds(LANES, LANES)] = zeros16

    iota = lax.iota(jnp.int32, LANES)

    def chunk_body(j, carry):
        row0 = j * LANES
        row_idx = row0 + iota
        cnt = jnp.zeros((LANES,), jnp.float32)
        accs = [jnp.zeros((LANES,), jnp.float32) for _ in range(EMBED_DIM)]
        for l in range(SEQ):
            tk = tok_v[l, pl.ds(row0, LANES)]
            cnt = cnt + (tk != 0).astype(jnp.float32)
            for d in range(EMBED_DIM):
                col = jnp.full((LANES,), d, jnp.int32)
                accs[d] = accs[d] + plsc.load_gather(ttab_v, [tk, col])
        rcp = 1.0 / jnp.maximum(cnt, 1e-9)
        for d in range(EMBED_DIM):
            col = jnp.full((LANES,), d, jnp.int32)
            plsc.store_scatter(pool_v, [row_idx, col], accs[d] * rcp)
        return carry

    lax.fori_loop(0, ROW_CHUNKS, chunk_body, 0)

    for g in gathers:
        g.wait()

    pltpu.sync_copy(title_rows_v, out_hbm.at[pl.ds(base, B_PER_W), pl.ds(0, EMBED_DIM)])
    pltpu.sync_copy(pool_v, out_hbm.at[pl.ds(base, B_PER_W), pl.ds(EMBED_DIM, EMBED_DIM)])


@jax.jit
def kernel(title_ids, token_ids, title_table, token_table):
    tid3 = title_ids.reshape(NUM_WORKERS, N_IDX_CHUNKS, IDX_CHUNK)
    tok3 = token_ids.reshape(NUM_WORKERS, B_PER_W, SEQ).transpose(0, 2, 1)

    run = pl.kernel(
        _body,
        out_type=jax.ShapeDtypeStruct((BATCH, 2 * EMBED_DIM), jnp.float32),
        mesh=plsc.VectorSubcoreMesh(core_axis_name="c", subcore_axis_name="s"),
        scratch_types=[
            pltpu.VMEM((N_IDX_CHUNKS, IDX_CHUNK), jnp.int32),
            pltpu.VMEM((B_PER_W, EMBED_DIM), jnp.float32),
            pltpu.VMEM((MAX_TOKENS, EMBED_DIM), jnp.float32),
            pltpu.VMEM((SEQ, B_PER_W), jnp.int32),
            pltpu.VMEM((B_PER_W, EMBED_DIM), jnp.float32),
            pltpu.SemaphoreType.DMA,
        ],
    )
    return run(tid3, tok3, title_table, token_table)


# speedup vs baseline: 2.4739x; 2.4739x over previous
import jax
import jax.numpy as jnp
from jax import lax
from jax.experimental import pallas as pl
from jax.experimental.pallas import tpu as pltpu
from jax.experimental.pallas import tpu_sc as plsc

VOCAB = 1000000
MAX_TOKENS = 1000
EMBED_DIM = 32
BATCH = 16384
SEQ = 20

NUM_CORES = 2
NUM_SUBCORES = 16
LANES = 16
NUM_WORKERS = NUM_CORES * NUM_SUBCORES
B_PER_W = BATCH // NUM_WORKERS
IDX_CHUNK = 128
N_IDX_CHUNKS = B_PER_W // IDX_CHUNK
ROW_CHUNKS = B_PER_W // LANES


def _body(tid_hbm, tok_hbm, title_tab_hbm, token_tab_hbm, out_hbm,
          tid_v, title_rows_v, ttab_v, tok_v, pool_v, gsem):
    wid = lax.axis_index("s") * NUM_CORES + lax.axis_index("c")
    base = wid * B_PER_W

    pltpu.sync_copy(tid_hbm.at[wid], tid_v)
    gathers = []
    for c in range(N_IDX_CHUNKS):
        gathers.append(
            pltpu.async_copy(
                title_tab_hbm.at[tid_v.at[c]],
                title_rows_v.at[pl.ds(c * IDX_CHUNK, IDX_CHUNK)],
                gsem,
            )
        )

    pltpu.sync_copy(token_tab_hbm, ttab_v)
    pltpu.sync_copy(tok_hbm.at[wid], tok_v)

    zeros16 = jnp.zeros((LANES,), jnp.float32)
    ttab_v[0, pl.ds(0, LANES)] = zeros16
    ttab_v[0, pl.ds(LANES, LANES)] = zeros16

    iota = lax.iota(jnp.int32, LANES)

    def chunk_body(j, carry):
        row0 = j * LANES
        row_idx = row0 + iota
        cnt = jnp.zeros((LANES,), jnp.float32)
        accs = [jnp.zeros((LANES,), jnp.float32) for _ in range(EMBED_DIM)]
        for l in range(SEQ):
            tk = tok_v[l, pl.ds(row0, LANES)]
            cnt = cnt + (tk != 0).astype(jnp.float32)
            for d in range(EMBED_DIM):
                col = jnp.full((LANES,), d, jnp.int32)
                accs[d] = accs[d] + plsc.load_gather(ttab_v, [tk, col])
        rcp = 1.0 / jnp.maximum(cnt, 1e-9)
        for d in range(EMBED_DIM):
            col = jnp.full((LANES,), d, jnp.int32)
            plsc.store_scatter(pool_v, [row_idx, col], accs[d] * rcp)
        return carry

    lax.fori_loop(0, ROW_CHUNKS, chunk_body, 0)

    for g in gathers:
        g.wait()

    pltpu.sync_copy(title_rows_v, out_hbm.at[0, pl.ds(base, B_PER_W), :])
    pltpu.sync_copy(pool_v, out_hbm.at[1, pl.ds(base, B_PER_W), :])


@jax.jit
def kernel(title_ids, token_ids, title_table, token_table):
    tid3 = title_ids.reshape(NUM_WORKERS, N_IDX_CHUNKS, IDX_CHUNK)
    tok3 = token_ids.reshape(NUM_WORKERS, B_PER_W, SEQ).transpose(0, 2, 1)

    run = pl.kernel(
        _body,
        out_type=jax.ShapeDtypeStruct((2, BATCH, EMBED_DIM), jnp.float32),
        mesh=plsc.VectorSubcoreMesh(core_axis_name="c", subcore_axis_name="s"),
        compiler_params=pltpu.CompilerParams(
            needs_layout_passes=False, use_tc_tiling_on_sc=False),
        scratch_types=[
            pltpu.VMEM((N_IDX_CHUNKS, IDX_CHUNK), jnp.int32),
            pltpu.VMEM((B_PER_W, EMBED_DIM), jnp.float32),
            pltpu.VMEM((MAX_TOKENS, EMBED_DIM), jnp.float32),
            pltpu.VMEM((SEQ, B_PER_W), jnp.int32),
            pltpu.VMEM((B_PER_W, EMBED_DIM), jnp.float32),
            pltpu.SemaphoreType.DMA,
        ],
    )
    halves = run(tid3, tok3, title_table, token_table)
    return halves.transpose(1, 0, 2).reshape(BATCH, 2 * EMBED_DIM)
